# baseline (speedup 1.0000x reference)
import functools

import jax
import jax.numpy as jnp
from jax import lax
from jax.experimental import pallas as pl
from jax.experimental.pallas import tpu as pltpu
from jax.experimental.pallas import tpu_sc as plsc

B, L, D = 1, 2048, 768
E, K, H = 8, 2, 1536
N = B * L
TILE = 256
T = 24
R = T * TILE
NW = 32
TOK_W = N // NW
CT = 32


def _gate_kernel(x_ref, scale_ref, bias_ref, gw_ref,
                 xn_ref, p1_ref, p2_ref, g1_ref, g2_ref, te_ref, bal_ref):
    x = x_ref[...]
    mu = jnp.mean(x, axis=1, keepdims=True)
    var = jnp.mean((x - mu) ** 2, axis=1, keepdims=True)
    xn = (x - mu) / jnp.sqrt(var + 1e-5) * scale_ref[...] + bias_ref[...]
    xn_ref[...] = xn
    logits = lax.dot_general(gw_ref[...], xn, (((1,), (1,)), ((), ())),
                             preferred_element_type=jnp.float32)
    iota_e = lax.broadcasted_iota(jnp.int32, (E, N), 0)
    m1 = jnp.max(logits, axis=0, keepdims=True)
    i1 = jnp.min(jnp.where(logits == m1, iota_e, E), axis=0, keepdims=True)
    masked = jnp.where(iota_e == i1, -jnp.inf, logits)
    m2 = jnp.max(masked, axis=0, keepdims=True)
    i2 = jnp.min(jnp.where(masked == m2, iota_e, E), axis=0, keepdims=True)
    e2 = jnp.exp(m2 - m1)
    denom = 1.0 + e2
    g1 = 1.0 / denom
    g2 = e2 / denom
    g1_ref[...] = g1
    g2_ref[...] = g2
    onehot1 = (iota_e == i1).astype(jnp.float32)
    onehot2 = (iota_e == i2).astype(jnp.float32)
    gates = onehot1 * g1 + onehot2 * g2
    load = jnp.mean(gates, axis=1, keepdims=True)
    bal_ref[...] = jnp.mean((load - 1.0 / E) ** 2).reshape(1, 1)

    lt = (lax.broadcasted_iota(jnp.int32, (N, N), 0)
          < lax.broadcasted_iota(jnp.int32, (N, N), 1)).astype(jnp.float32)
    rank1 = lax.dot_general(onehot1, lt, (((1,), (0,)), ((), ())),
                            preferred_element_type=jnp.float32)
    rank2 = lax.dot_general(onehot2, lt, (((1,), (0,)), ((), ())),
                            preferred_element_type=jnp.float32)
    cnt1 = jnp.sum(onehot1, axis=1, keepdims=True)
    cnt2 = jnp.sum(onehot2, axis=1, keepdims=True)
    tiles = jnp.floor((cnt1 + cnt2 + float(TILE - 1)) * (1.0 / TILE))
    lte = (lax.broadcasted_iota(jnp.int32, (E, E), 1)
           < lax.broadcasted_iota(jnp.int32, (E, E), 0)).astype(jnp.float32)
    tile_off = lax.dot_general(lte, tiles, (((1,), (0,)), ((), ())),
                               preferred_element_type=jnp.float32)
    row_off = tile_off * float(TILE)
    pos1 = jnp.sum(onehot1 * (row_off + rank1), axis=0, keepdims=True)
    pos2 = jnp.sum(onehot2 * (row_off + cnt1 + rank2), axis=0, keepdims=True)
    p1_ref[...] = pos1.astype(jnp.int32)
    p2_ref[...] = pos2.astype(jnp.int32)
    tile_end = tile_off + tiles
    iota_t = lax.broadcasted_iota(jnp.int32, (E, T), 1).astype(jnp.float32)
    te = jnp.sum((iota_t >= tile_end).astype(jnp.float32), axis=0,
                 keepdims=True)
    te_ref[...] = jnp.minimum(te, float(E - 1)).astype(jnp.int32)


def _expert_kernel(te_ref, xs_ref, w1_ref, b1_ref, w2_ref, b2_ref, oe_ref):
    xs = xs_ref[...]
    h = lax.dot_general(xs, w1_ref[0], (((1,), (1,)), ((), ())),
                        preferred_element_type=jnp.float32) + b1_ref[0]
    h = 0.5 * h * (1.0 + lax.erf(h * 0.7071067811865476))
    oe_ref[...] = lax.dot_general(h, w2_ref[0], (((1,), (1,)), ((), ())),
                                  preferred_element_type=jnp.float32) + b2_ref[0]


_sc_mesh = plsc.VectorSubcoreMesh(core_axis_name="c", subcore_axis_name="s")


@functools.partial(
    pl.kernel,
    mesh=_sc_mesh,
    out_type=jax.ShapeDtypeStruct((R, D), jnp.float32),
    scratch_types=[
        pltpu.VMEM((TOK_W, D), jnp.float32),
        pltpu.VMEM((TOK_W,), jnp.int32),
        pltpu.VMEM((TOK_W,), jnp.int32),
        pltpu.SemaphoreType.DMA,
        pltpu.SemaphoreType.DMA,
    ],
)
def _dispatch(xn_hbm, p1_hbm, p2_hbm, xs_hbm,
              xrow_v, p1_v, p2_v, sem1, sem2):
    wid = lax.axis_index("s") * 2 + lax.axis_index("c")
    base = wid * TOK_W
    pltpu.sync_copy(p1_hbm.at[pl.ds(base, TOK_W)], p1_v)
    pltpu.sync_copy(p2_hbm.at[pl.ds(base, TOK_W)], p2_v)
    pltpu.sync_copy(xn_hbm.at[pl.ds(base, TOK_W)], xrow_v)
    c1 = pltpu.async_copy(xrow_v, xs_hbm.at[p1_v], sem1)
    c2 = pltpu.async_copy(xrow_v, xs_hbm.at[p2_v], sem2)
    c1.wait()
    c2.wait()


@functools.partial(
    pl.kernel,
    mesh=_sc_mesh,
    out_type=jax.ShapeDtypeStruct((N, D), jnp.float32),
    scratch_types=[
        pltpu.VMEM((CT, D), jnp.float32),
        pltpu.VMEM((CT, D), jnp.float32),
        pltpu.VMEM((CT, D), jnp.float32),
        pltpu.VMEM((CT,), jnp.int32),
        pltpu.VMEM((CT,), jnp.int32),
        pltpu.VMEM((CT, 16), jnp.float32),
        pltpu.VMEM((CT, 16), jnp.float32),
        pltpu.SemaphoreType.DMA,
        pltpu.SemaphoreType.DMA,
    ],
)
def _combine(x_hbm, oe_hbm, p1_hbm, p2_hbm, g1r_hbm, g2r_hbm, out_hbm,
             acc_v, o1_v, o2_v, p1_v, p2_v, g1r_v, g2r_v, sem1, sem2):
    wid = lax.axis_index("s") * 2 + lax.axis_index("c")
    for c in range(TOK_W // CT):
        base = wid * TOK_W + c * CT
        pltpu.sync_copy(p1_hbm.at[pl.ds(base, CT)], p1_v)
        pltpu.sync_copy(p2_hbm.at[pl.ds(base, CT)], p2_v)
        pltpu.sync_copy(x_hbm.at[pl.ds(base, CT)], acc_v)
        pltpu.sync_copy(g1r_hbm.at[pl.ds(base, CT)], g1r_v)
        pltpu.sync_copy(g2r_hbm.at[pl.ds(base, CT)], g2r_v)
        c1 = pltpu.async_copy(oe_hbm.at[p1_v], o1_v, sem1)
        c2 = pltpu.async_copy(oe_hbm.at[p2_v], o2_v, sem2)
        c1.wait()
        c2.wait()

        def body(i, _):
            gv1 = g1r_v[i, pl.ds(0, 16)]
            gv2 = g2r_v[i, pl.ds(0, 16)]
            for k in range(D // 16):
                sl = pl.ds(k * 16, 16)
                acc_v[i, sl] = (acc_v[i, sl] + gv1 * o1_v[i, sl]
                                + gv2 * o2_v[i, sl])
            return 0

        lax.fori_loop(0, CT, body, 0)
        pltpu.sync_copy(acc_v, out_hbm.at[pl.ds(base, CT)])


def kernel(x, norm_scale, norm_bias, gate_w, W1, B1, W2, B2):
    xf = x.reshape(N, D)
    xn, p1, p2, g1, g2, te, bal = pl.pallas_call(
        _gate_kernel,
        out_shape=[
            jax.ShapeDtypeStruct((N, D), jnp.float32),
            jax.ShapeDtypeStruct((1, N), jnp.int32),
            jax.ShapeDtypeStruct((1, N), jnp.int32),
            jax.ShapeDtypeStruct((1, N), jnp.float32),
            jax.ShapeDtypeStruct((1, N), jnp.float32),
            jax.ShapeDtypeStruct((1, T), jnp.int32),
            jax.ShapeDtypeStruct((1, 1), jnp.float32),
        ],
    )(xf, norm_scale.reshape(1, D), norm_bias.reshape(1, D), gate_w)

    p1f = p1.reshape(N)
    p2f = p2.reshape(N)
    xs = _dispatch(xn, p1f, p2f)

    grid_spec = pltpu.PrefetchScalarGridSpec(
        num_scalar_prefetch=1,
        grid=(T,),
        in_specs=[
            pl.BlockSpec((TILE, D), lambda t, te: (t, 0)),
            pl.BlockSpec((1, H, D), lambda t, te: (te[t], 0, 0)),
            pl.BlockSpec((1, 1, H), lambda t, te: (te[t], 0, 0)),
            pl.BlockSpec((1, D, H), lambda t, te: (te[t], 0, 0)),
            pl.BlockSpec((1, 1, D), lambda t, te: (te[t], 0, 0)),
        ],
        out_specs=pl.BlockSpec((TILE, D), lambda t, te: (t, 0)),
    )
    oe = pl.pallas_call(
        _expert_kernel,
        grid_spec=grid_spec,
        out_shape=jax.ShapeDtypeStruct((R, D), jnp.float32),
    )(te.reshape(T), xs, W1, B1.reshape(E, 1, H), W2, B2.reshape(E, 1, D))

    g1r = jnp.broadcast_to(g1.reshape(N, 1), (N, 16))
    g2r = jnp.broadcast_to(g2.reshape(N, 1), (N, 16))
    out = _combine(xf, oe, p1f, p2f, g1r, g2r)
    return out.reshape(B, L, D), bal.reshape(())
